# baseline (speedup 1.0000x reference)
import jax
import jax.numpy as jnp
import numpy as np
from jax import lax
from jax.experimental import pallas as pl
from jax.experimental.pallas import tpu as pltpu
from jax.experimental.pallas import tpu_sc as plsc

N = 10000
E = 320000
D = 128
H1 = 64
H2 = 128
HFC = 256

NC = 2
NS = 16
LANES = 16
NW = NC * NS
EPW = E // NW
CH = 80
NCH = EPW // CH
RPT = N // NS

BLK = 400


def _lh_perm(h):
    lo = (np.arange(0, h, 32)[:, None] + np.arange(16)).reshape(-1)
    return np.concatenate([lo, lo + 16])


_PERM1 = _lh_perm(H1)
_PERM2 = _lh_perm(H2)


def _pack_rows(t, h):
    lo = t[:, :h // 2].astype(jnp.bfloat16).astype(jnp.float32)
    hi = t[:, h // 2:].astype(jnp.bfloat16).astype(jnp.float32)
    ulo = lax.shift_right_logical(
        lax.bitcast_convert_type(lo, jnp.uint32), jnp.uint32(16))
    uhi = lax.bitcast_convert_type(hi, jnp.uint32) & jnp.uint32(0xFFFF0000)
    return lax.bitcast_convert_type(ulo | uhi, jnp.int32)


def _make_sc_edge(h, with_count):
    hm = h + 2 * LANES if with_count else h
    SL = 125
    mesh = plsc.VectorSubcoreMesh(core_axis_name="c", subcore_axis_name="s")
    out_type = jax.ShapeDtypeStruct((NC, N, hm), jnp.float32)
    scratch = [
        pltpu.VMEM((NCH, CH), jnp.int32),
        pltpu.VMEM((NCH, CH), jnp.int32),
        pltpu.VMEM((CH, h // 2), jnp.int32),
        pltpu.VMEM((CH, h // 2), jnp.int32),
        pltpu.VMEM((CH, h // 2), jnp.int32),
        pltpu.VMEM((CH, h // 2), jnp.int32),
        pltpu.VMEM((CH, hm), jnp.bfloat16),
        pltpu.VMEM((CH, hm), jnp.bfloat16),
        pltpu.VMEM_SHARED((N, hm), jnp.bfloat16),
        pltpu.SemaphoreType.DMA,
        pltpu.SemaphoreType.DMA,
        pltpu.SemaphoreType.DMA,
        pltpu.SemaphoreType.DMA,
        pltpu.SemaphoreType.DMA,
        pltpu.SemaphoreType.DMA,
        pltpu.VMEM((SL, hm), jnp.bfloat16),
        pltpu.VMEM((SL, hm), jnp.float32),
    ]

    def body(t_hbm, dst_hbm, src_hbm, z_hbm, out_hbm,
             dst_v, src_v, a0, a1, b0, b1, m0, m1, acc_s,
             sa0, sa1, sb0, sb1, ss0, ss1, cbuf, obuf):
        cid = lax.axis_index("c")
        sid = lax.axis_index("s")
        wid = cid * NS + sid
        r0 = sid * RPT
        pltpu.sync_copy(z_hbm.at[pl.ds(r0, RPT)], acc_s.at[pl.ds(r0, RPT)])
        pltpu.sync_copy(dst_hbm.at[wid], dst_v)
        pltpu.sync_copy(src_hbm.at[wid], src_v)
        plsc.subcore_barrier()

        if with_count:
            ones = jnp.full((2 * LANES,), 1.0, jnp.bfloat16)

        def gather(c, av, bv, sa, sb):
            pltpu.async_copy(t_hbm.at[dst_v.at[c]], av, sa)
            pltpu.async_copy(t_hbm.at[src_v.at[c]], bv, sb)

        def wait_gather(av, bv, sa, sb):
            pltpu.make_async_copy(t_hbm.at[dst_v.at[0]], av, sa).wait()
            pltpu.make_async_copy(t_hbm.at[src_v.at[0]], bv, sb).wait()

        def dsti(c):
            return dst_v.at[c]

        def compute(av, bv, mv):
            zero = jnp.bfloat16(0.0)

            def edge(e, _):
                for k in range(h // 32):
                    s = 32 * k
                    va = plsc.bitcast(av[e, pl.ds(s // 2, LANES)],
                                      jnp.bfloat16)
                    vb = plsc.bitcast(bv[e, pl.ds(s // 2, LANES)],
                                      jnp.bfloat16)
                    mv[e, pl.ds(s, 2 * LANES)] = jnp.maximum(va + vb, zero)
                if with_count:
                    mv[e, pl.ds(h, 2 * LANES)] = ones
                return 0

            lax.fori_loop(0, CH, edge, 0, unroll=8)

        def scatter(c, mv, ss):
            pltpu.async_copy(mv, acc_s.at[dsti(c)], ss, add=True)

        def wait_scatter(mv, ss):
            pltpu.make_async_copy(mv, acc_s.at[dsti(0)], ss).wait()

        gather(0, a0, b0, sa0, sb0)

        def it(i, _):
            c0 = 2 * i
            gather(c0 + 1, a1, b1, sa1, sb1)
            wait_gather(a0, b0, sa0, sb0)

            @pl.when(i > 0)
            def _():
                wait_scatter(m0, ss0)

            compute(a0, b0, m0)
            scatter(c0, m0, ss0)
            gather(c0 + 2, a0, b0, sa0, sb0)
            wait_gather(a1, b1, sa1, sb1)

            @pl.when(i > 0)
            def _():
                wait_scatter(m1, ss1)

            compute(a1, b1, m1)
            scatter(c0 + 1, m1, ss1)
            return 0

        lax.fori_loop(0, (NCH - 1) // 2, it, 0)
        wait_gather(a0, b0, sa0, sb0)
        wait_scatter(m0, ss0)
        compute(a0, b0, m0)
        scatter(NCH - 1, m0, ss0)
        wait_scatter(m0, ss0)
        wait_scatter(m1, ss1)
        plsc.subcore_barrier()
        for t in range(RPT // SL):
            rr = r0 + t * SL
            pltpu.sync_copy(acc_s.at[pl.ds(rr, SL)], cbuf)

            def conv(r2, _):
                for k in range(hm // 32):
                    s = 32 * k
                    lo, hi = plsc.unpack(
                        cbuf[r2, pl.ds(s, 2 * LANES)],
                        format=plsc.PackFormat.INTERLEAVED)
                    obuf[r2, pl.ds(s, LANES)] = lo
                    obuf[r2, pl.ds(s + LANES, LANES)] = hi
                return 0

            lax.fori_loop(0, SL, conv, 0, unroll=4)
            pltpu.sync_copy(obuf, out_hbm.at[cid, pl.ds(rr, SL)])

    return pl.kernel(body, out_type=out_type, mesh=mesh,
                     scratch_types=scratch,
                     compiler_params=pltpu.CompilerParams(
                         use_tc_tiling_on_sc=False,
                         needs_layout_passes=False))


def _tc1_body(x_ref, w_ref, b_ref, t_ref):
    x = x_ref[...]
    w = w_ref[...]
    wa = w[:, :D]
    wb = w[:, D:]
    dn = (((1,), (1,)), ((), ()))
    t_ref[0] = _pack_rows(lax.dot_general(x, wa - wb, dn) + b_ref[...], H1)
    t_ref[1] = _pack_rows(lax.dot_general(x, wb, dn), H1)


def _tc2_body(p_ref, w_ref, b_ref, t_ref):
    s = p_ref[0] + p_ref[1]
    cnt = s[:, H1]
    hn = s[:, :H1] / jnp.maximum(cnt, 1.0)[:, None]
    w = w_ref[...]
    wa = w[:, :H1]
    wb = w[:, H1:]
    dn = (((1,), (1,)), ((), ()))
    t_ref[0] = _pack_rows(lax.dot_general(hn, wa - wb, dn) + b_ref[...], H2)
    t_ref[1] = _pack_rows(lax.dot_general(hn, wb, dn), H2)


def _tc3_body(p_ref, c_ref, w3_ref, b3_ref, w4_ref, b4_ref, o_ref):
    s = p_ref[0] + p_ref[1]
    c = c_ref[0] + c_ref[1]
    cnt = c[:, H1]
    hn = s / jnp.maximum(cnt, 1.0)[:, None]
    dn = (((1,), (1,)), ((), ()))
    f = jnp.maximum(lax.dot_general(hn, w3_ref[...], dn) + b3_ref[...], 0.0)
    o = jnp.sum(f * w4_ref[...], axis=1, keepdims=True) + b4_ref[0, 0]
    o_ref[...] = jax.nn.sigmoid(o)


def kernel(X, edge_index, W1, b1, W2, b2, W3, b3, W4, b4):
    f32 = jnp.float32
    bf16 = jnp.bfloat16
    W1p = W1[_PERM1]
    b1p = b1[_PERM1]
    W2p = W2[_PERM2]
    b2p = b2[_PERM2]
    dstc = edge_index[1].reshape(NW, NCH, CH)
    srcc = edge_index[0].reshape(NW, NCH, CH) + N
    z1 = jnp.zeros((N, H1 + 2 * LANES), jnp.bfloat16)
    z2 = jnp.zeros((N, H2), jnp.bfloat16)

    grid = (N // BLK,)
    full = lambda shape: pl.BlockSpec(shape, lambda i: (0,) * len(shape))
    rows = lambda w: pl.BlockSpec((BLK, w), lambda i: (i, 0))
    two = lambda w: pl.BlockSpec((2, BLK, w), lambda i: (0, i, 0))

    T1 = pl.pallas_call(
        _tc1_body,
        grid=grid,
        in_specs=[rows(D), full((H1, 2 * D)), full((1, H1))],
        out_specs=two(H1 // 2),
        out_shape=jax.ShapeDtypeStruct((2, N, H1 // 2), jnp.int32),
    )(X, W1p, b1p.reshape(1, H1))

    P1 = _make_sc_edge(H1, True)(T1.reshape(2 * N, H1 // 2), dstc, srcc, z1)

    T2 = pl.pallas_call(
        _tc2_body,
        grid=grid,
        in_specs=[two(H1 + 2 * LANES), full((H2, 2 * H1)), full((1, H2))],
        out_specs=two(H2 // 2),
        out_shape=jax.ShapeDtypeStruct((2, N, H2 // 2), jnp.int32),
    )(P1, W2p, b2p.reshape(1, H2))

    P2 = _make_sc_edge(H2, False)(T2.reshape(2 * N, H2 // 2), dstc, srcc,
                                  z2)

    out = pl.pallas_call(
        _tc3_body,
        grid=grid,
        in_specs=[two(H2), two(H1 + 2 * LANES), full((HFC, H2)), full((1, HFC)),
                  full((1, HFC)), full((1, 1))],
        out_specs=rows(1),
        out_shape=jax.ShapeDtypeStruct((N, 1), f32),
    )(P2, P1, W3, b3.reshape(1, HFC), W4, b4.reshape(1, 1))

    return out.reshape(N)


# speedup vs baseline: 1.0033x; 1.0033x over previous
import jax
import jax.numpy as jnp
import numpy as np
from jax import lax
from jax.experimental import pallas as pl
from jax.experimental.pallas import tpu as pltpu
from jax.experimental.pallas import tpu_sc as plsc

N = 10000
E = 320000
D = 128
H1 = 64
H2 = 128
HFC = 256

NC = 2
NS = 16
LANES = 16
NW = NC * NS
EPW = E // NW
CH = 80
NCH = EPW // CH
RPT = N // NS

BLK = 400


def _lh_perm(h):
    lo = (np.arange(0, h, 32)[:, None] + np.arange(16)).reshape(-1)
    return np.concatenate([lo, lo + 16])


_PERM1 = _lh_perm(H1)
_PERM2 = _lh_perm(H2)


def _pack_rows(t, h):
    lo = t[:, :h // 2].astype(jnp.bfloat16).astype(jnp.float32)
    hi = t[:, h // 2:].astype(jnp.bfloat16).astype(jnp.float32)
    ulo = lax.shift_right_logical(
        lax.bitcast_convert_type(lo, jnp.uint32), jnp.uint32(16))
    uhi = lax.bitcast_convert_type(hi, jnp.uint32) & jnp.uint32(0xFFFF0000)
    return lax.bitcast_convert_type(ulo | uhi, jnp.int32)


def _make_sc_edge(h, with_count):
    hm = h + 2 * LANES if with_count else h
    SL = 125
    mesh = plsc.VectorSubcoreMesh(core_axis_name="c", subcore_axis_name="s")
    out_type = jax.ShapeDtypeStruct((NC, N, hm), jnp.float32)
    scratch = [
        pltpu.VMEM((NCH, CH), jnp.int32),
        pltpu.VMEM((NCH, CH), jnp.int32),
        pltpu.VMEM((CH, h // 2), jnp.int32),
        pltpu.VMEM((CH, h // 2), jnp.int32),
        pltpu.VMEM((CH, h // 2), jnp.int32),
        pltpu.VMEM((CH, h // 2), jnp.int32),
        pltpu.VMEM((CH, hm), jnp.bfloat16),
        pltpu.VMEM((CH, hm), jnp.bfloat16),
        pltpu.VMEM_SHARED((N, hm), jnp.bfloat16),
        pltpu.SemaphoreType.DMA,
        pltpu.SemaphoreType.DMA,
        pltpu.SemaphoreType.DMA,
        pltpu.SemaphoreType.DMA,
        pltpu.SemaphoreType.DMA,
        pltpu.SemaphoreType.DMA,
        pltpu.VMEM((SL, hm), jnp.bfloat16),
        pltpu.VMEM((SL, hm), jnp.float32),
    ]

    def body(t_hbm, dst_hbm, src_hbm, z_hbm, out_hbm,
             dst_v, src_v, a0, a1, b0, b1, m0, m1, acc_s,
             sa0, sa1, sb0, sb1, ss0, ss1, cbuf, obuf):
        cid = lax.axis_index("c")
        sid = lax.axis_index("s")
        wid = cid * NS + sid
        r0 = sid * RPT
        pltpu.sync_copy(z_hbm.at[pl.ds(r0, RPT)], acc_s.at[pl.ds(r0, RPT)])
        pltpu.sync_copy(dst_hbm.at[wid], dst_v)
        pltpu.sync_copy(src_hbm.at[wid], src_v)
        plsc.subcore_barrier()

        if with_count:
            ones = jnp.full((2 * LANES,), 1.0, jnp.bfloat16)

        def gather(c, av, bv, sa, sb):
            pltpu.async_copy(t_hbm.at[dst_v.at[c]], av, sa)
            pltpu.async_copy(t_hbm.at[src_v.at[c]], bv, sb)

        def wait_gather(av, bv, sa, sb):
            pltpu.make_async_copy(t_hbm.at[dst_v.at[0]], av, sa).wait()
            pltpu.make_async_copy(t_hbm.at[src_v.at[0]], bv, sb).wait()

        def dsti(c):
            return dst_v.at[c]

        def compute(av, bv, mv):
            zero = jnp.bfloat16(0.0)

            def edge(e, _):
                for k in range(h // 32):
                    s = 32 * k
                    va = plsc.bitcast(av[e, pl.ds(s // 2, LANES)],
                                      jnp.bfloat16)
                    vb = plsc.bitcast(bv[e, pl.ds(s // 2, LANES)],
                                      jnp.bfloat16)
                    mv[e, pl.ds(s, 2 * LANES)] = jnp.maximum(va + vb, zero)
                if with_count:
                    mv[e, pl.ds(h, 2 * LANES)] = ones
                return 0

            lax.fori_loop(0, CH, edge, 0, unroll=4)

        def scatter(c, mv, ss):
            pltpu.async_copy(mv, acc_s.at[dsti(c)], ss, add=True)

        def wait_scatter(mv, ss):
            pltpu.make_async_copy(mv, acc_s.at[dsti(0)], ss).wait()

        gather(0, a0, b0, sa0, sb0)

        def it(i, _):
            c0 = 2 * i
            gather(c0 + 1, a1, b1, sa1, sb1)
            wait_gather(a0, b0, sa0, sb0)

            @pl.when(i > 0)
            def _():
                wait_scatter(m0, ss0)

            compute(a0, b0, m0)
            scatter(c0, m0, ss0)
            gather(c0 + 2, a0, b0, sa0, sb0)
            wait_gather(a1, b1, sa1, sb1)

            @pl.when(i > 0)
            def _():
                wait_scatter(m1, ss1)

            compute(a1, b1, m1)
            scatter(c0 + 1, m1, ss1)
            return 0

        lax.fori_loop(0, (NCH - 1) // 2, it, 0)
        wait_gather(a0, b0, sa0, sb0)
        wait_scatter(m0, ss0)
        compute(a0, b0, m0)
        scatter(NCH - 1, m0, ss0)
        wait_scatter(m0, ss0)
        wait_scatter(m1, ss1)
        plsc.subcore_barrier()
        for t in range(RPT // SL):
            rr = r0 + t * SL
            pltpu.sync_copy(acc_s.at[pl.ds(rr, SL)], cbuf)

            def conv(r2, _):
                for k in range(hm // 32):
                    s = 32 * k
                    lo, hi = plsc.unpack(
                        cbuf[r2, pl.ds(s, 2 * LANES)],
                        format=plsc.PackFormat.INTERLEAVED)
                    obuf[r2, pl.ds(s, LANES)] = lo
                    obuf[r2, pl.ds(s + LANES, LANES)] = hi
                return 0

            lax.fori_loop(0, SL, conv, 0, unroll=4)
            pltpu.sync_copy(obuf, out_hbm.at[cid, pl.ds(rr, SL)])

    return pl.kernel(body, out_type=out_type, mesh=mesh,
                     scratch_types=scratch,
                     compiler_params=pltpu.CompilerParams(
                         use_tc_tiling_on_sc=False,
                         needs_layout_passes=False))


def _tc1_body(x_ref, w_ref, b_ref, t_ref):
    x = x_ref[...]
    w = w_ref[...]
    wa = w[:, :D]
    wb = w[:, D:]
    dn = (((1,), (1,)), ((), ()))
    t_ref[0] = _pack_rows(lax.dot_general(x, wa - wb, dn) + b_ref[...], H1)
    t_ref[1] = _pack_rows(lax.dot_general(x, wb, dn), H1)


def _tc2_body(p_ref, w_ref, b_ref, t_ref):
    s = p_ref[0] + p_ref[1]
    cnt = s[:, H1]
    hn = s[:, :H1] / jnp.maximum(cnt, 1.0)[:, None]
    w = w_ref[...]
    wa = w[:, :H1]
    wb = w[:, H1:]
    dn = (((1,), (1,)), ((), ()))
    t_ref[0] = _pack_rows(lax.dot_general(hn, wa - wb, dn) + b_ref[...], H2)
    t_ref[1] = _pack_rows(lax.dot_general(hn, wb, dn), H2)


def _tc3_body(p_ref, c_ref, w3_ref, b3_ref, w4_ref, b4_ref, o_ref):
    s = p_ref[0] + p_ref[1]
    c = c_ref[0] + c_ref[1]
    cnt = c[:, H1]
    hn = s / jnp.maximum(cnt, 1.0)[:, None]
    dn = (((1,), (1,)), ((), ()))
    f = jnp.maximum(lax.dot_general(hn, w3_ref[...], dn) + b3_ref[...], 0.0)
    o = jnp.sum(f * w4_ref[...], axis=1, keepdims=True) + b4_ref[0, 0]
    o_ref[...] = jax.nn.sigmoid(o)


def kernel(X, edge_index, W1, b1, W2, b2, W3, b3, W4, b4):
    f32 = jnp.float32
    bf16 = jnp.bfloat16
    W1p = W1[_PERM1]
    b1p = b1[_PERM1]
    W2p = W2[_PERM2]
    b2p = b2[_PERM2]
    dstc = edge_index[1].reshape(NW, NCH, CH)
    srcc = edge_index[0].reshape(NW, NCH, CH) + N
    z1 = jnp.zeros((N, H1 + 2 * LANES), jnp.bfloat16)
    z2 = jnp.zeros((N, H2), jnp.bfloat16)

    grid = (N // BLK,)
    full = lambda shape: pl.BlockSpec(shape, lambda i: (0,) * len(shape))
    rows = lambda w: pl.BlockSpec((BLK, w), lambda i: (i, 0))
    two = lambda w: pl.BlockSpec((2, BLK, w), lambda i: (0, i, 0))

    T1 = pl.pallas_call(
        _tc1_body,
        grid=grid,
        in_specs=[rows(D), full((H1, 2 * D)), full((1, H1))],
        out_specs=two(H1 // 2),
        out_shape=jax.ShapeDtypeStruct((2, N, H1 // 2), jnp.int32),
    )(X, W1p, b1p.reshape(1, H1))

    P1 = _make_sc_edge(H1, True)(T1.reshape(2 * N, H1 // 2), dstc, srcc, z1)

    T2 = pl.pallas_call(
        _tc2_body,
        grid=grid,
        in_specs=[two(H1 + 2 * LANES), full((H2, 2 * H1)), full((1, H2))],
        out_specs=two(H2 // 2),
        out_shape=jax.ShapeDtypeStruct((2, N, H2 // 2), jnp.int32),
    )(P1, W2p, b2p.reshape(1, H2))

    P2 = _make_sc_edge(H2, False)(T2.reshape(2 * N, H2 // 2), dstc, srcc,
                                  z2)

    out = pl.pallas_call(
        _tc3_body,
        grid=grid,
        in_specs=[two(H2), two(H1 + 2 * LANES), full((HFC, H2)), full((1, HFC)),
                  full((1, HFC)), full((1, 1))],
        out_specs=rows(1),
        out_shape=jax.ShapeDtypeStruct((N, 1), f32),
    )(P2, P1, W3, b3.reshape(1, HFC), W4, b4.reshape(1, 1))

    return out.reshape(N)
